# Initial kernel scaffold; baseline (speedup 1.0000x reference)
import jax
import jax.numpy as jnp
from jax.experimental import pallas as pl


def kernel(x, W):
    raise NotImplementedError("write your pallas kernel here")


# speedup vs baseline: 408.3937x; 408.3937x over previous
import functools

import jax
import jax.numpy as jnp
from jax import lax
from jax.experimental import pallas as pl
from jax.experimental.pallas import tpu as pltpu
from jax.experimental.pallas import tpu_sc as plsc

_NT = 60
_PS = 3375
_PP = 3376
_B = 16384
_NC = 2
_NS = 16
_TG = 4
_BG = 8
_TPG = _NT // _TG
_BPG = _B // _BG
_TAB_W = _TPG * _PP
_LANES = 16


def _sc_body(xT_hbm, tab_hbm, out_hbm, tab_v, x_v, acc_v, tmp_v, shared):
    c = lax.axis_index("c")
    s = lax.axis_index("s")
    tg = s % _TG
    bg = c * (_NS // _TG) + s // _TG

    pltpu.sync_copy(tab_hbm.at[tg], tab_v)
    pltpu.sync_copy(xT_hbm.at[tg, :, pl.ds(bg * _BPG, _BPG)], x_v)

    def body(v, _):
        pos = pl.multiple_of(v * _LANES, _LANES)
        acc = jnp.zeros((_LANES,), jnp.float32)
        for k in range(_TPG):
            xv = x_v[k, pl.ds(pos, _LANES)]
            acc = acc + plsc.load_gather(tab_v, [xv + (k * _PP)])
        acc_v[pl.ds(pos, _LANES)] = acc
        return 0

    lax.fori_loop(0, _BPG // _LANES, body, 0)

    pltpu.sync_copy(acc_v, shared.at[s])
    plsc.subcore_barrier()

    @pl.when(tg == 0)
    def _():
        for j in range(1, _TG):
            pltpu.sync_copy(shared.at[s + j], tmp_v.at[j - 1])

        def red(v, _):
            pos = pl.multiple_of(v * _LANES, _LANES)
            tot = acc_v[pl.ds(pos, _LANES)]
            for j in range(_TG - 1):
                tot = tot + tmp_v[j, pl.ds(pos, _LANES)]
            acc_v[pl.ds(pos, _LANES)] = tot
            return 0

        lax.fori_loop(0, _BPG // _LANES, red, 0)
        pltpu.sync_copy(acc_v, out_hbm.at[pl.ds(bg * _BPG, _BPG)])


@jax.jit
def _sc_call(xT, tab_flat):
    mesh = plsc.VectorSubcoreMesh(
        core_axis_name="c", subcore_axis_name="s",
        num_cores=_NC, num_subcores=_NS)
    f = pl.kernel(
        _sc_body,
        out_type=jax.ShapeDtypeStruct((_B,), jnp.float32),
        mesh=mesh,
        scratch_types=[
            pltpu.VMEM((_TAB_W,), jnp.float32),
            pltpu.VMEM((_TPG, _BPG), jnp.int32),
            pltpu.VMEM((_BPG,), jnp.float32),
            pltpu.VMEM((_TG - 1, _BPG), jnp.float32),
            pltpu.VMEM_SHARED((_NS, _BPG), jnp.float32),
        ],
        compiler_params=pltpu.CompilerParams(
            use_tc_tiling_on_sc=False, needs_layout_passes=False),
    )
    return f(xT, tab_flat)


def kernel(x, W):
    tab = jnp.pad(W[:, :, 0], ((0, 0), (0, _PP - _PS))).reshape(_TG, _TAB_W)
    xT = x.T.astype(jnp.int32).reshape(_TG, _TPG, _B)
    y = _sc_call(xT, tab)
    return y[:, None]
